# baseline (speedup 1.0000x reference)
import functools

import jax
import jax.numpy as jnp
from jax import lax
from jax.experimental import pallas as pl
from jax.experimental.pallas import tpu as pltpu
from jax.experimental.pallas import tpu_sc as plsc

SIZE = 32
V = SIZE * SIZE
C = 256
B = 64
NB = 8
L = 16
NW = 32
ROWS_PER_W = B // NW


def _tc_kernel(a_ref, x_ref, lp_ref):
    rows = []
    for i in range(NB):
        Xi = x_ref[i]
        ai = a_ref[i, :].reshape(1, C)
        rows.append(lax.dot_general(
            ai, Xi, (((1,), (1,)), ((), ()))))
    xc = jnp.concatenate(rows, axis=0)
    m = jnp.max(xc, axis=1, keepdims=True)
    lse = jnp.log(jnp.sum(jnp.exp(xc - m), axis=1, keepdims=True)) + m
    lp_ref[...] = xc - lse


def _sc_sample(lp_hbm, g_hbm, arg_hbm, lpv_hbm, lp_v, g_v, sv, si, st_i, sem):
    wid = lax.axis_index("s") * 2 + lax.axis_index("c")
    lanes = lax.iota(jnp.int32, L)
    for t in range(ROWS_PER_W):
        r = wid * ROWS_PER_W + t
        pltpu.async_copy(lp_hbm.at[r], lp_v, sem).wait()
        pltpu.async_copy(g_hbm.at[r], g_v, sem).wait()
        best_v = lp_v[pl.ds(0, L)] + g_v[pl.ds(0, L)]
        best_i = lanes
        for j in range(1, V // L):
            v = lp_v[pl.ds(j * L, L)] + g_v[pl.ds(j * L, L)]
            upd = v > best_v
            best_v = jnp.where(upd, v, best_v)
            best_i = jnp.where(upd, lanes + j * L, best_i)
        for off in (8, 4, 2, 1):
            sv[...] = best_v
            si[...] = best_i
            pidx = lanes ^ off
            pv = plsc.load_gather(sv, [pidx])
            pi = plsc.load_gather(si, [pidx])
            take = (pv > best_v) | ((pv == best_v) & (pi < best_i))
            best_v = jnp.where(take, pv, best_v)
            best_i = jnp.where(take, pi, best_i)
        lpv_vec = plsc.load_gather(lp_v, [best_i])
        xv = best_i & (SIZE - 1)
        yv = best_i >> 5
        st_i[...] = jnp.where(lanes == 0, xv,
                              jnp.where(lanes == 1, yv, 0))
        pltpu.sync_copy(st_i, arg_hbm.at[r])
        sv[...] = lpv_vec
        pltpu.sync_copy(sv, lpv_hbm.at[r])


def kernel(x, embedded_a):
    xt = x.transpose(0, 2, 3, 1).reshape(B, V, C)
    g = jax.random.gumbel(jax.random.key(42), (B, V), dtype=jnp.float32)
    lp = pl.pallas_call(
        _tc_kernel,
        grid=(B // NB,),
        in_specs=[
            pl.BlockSpec((NB, C), lambda b: (b, 0)),
            pl.BlockSpec((NB, V, C), lambda b: (b, 0, 0)),
        ],
        out_specs=pl.BlockSpec((NB, V), lambda b: (b, 0)),
        out_shape=jax.ShapeDtypeStruct((B, V), jnp.float32),
        compiler_params=pltpu.CompilerParams(
            dimension_semantics=("arbitrary",),
        ),
    )(embedded_a, xt)

    mesh = plsc.VectorSubcoreMesh(core_axis_name="c", subcore_axis_name="s")
    sampler = functools.partial(
        pl.kernel, mesh=mesh,
        out_type=[
            jax.ShapeDtypeStruct((B, L), jnp.int32),
            jax.ShapeDtypeStruct((B, L), jnp.float32),
        ],
        scratch_types=[
            pltpu.VMEM((V,), jnp.float32),
            pltpu.VMEM((V,), jnp.float32),
            pltpu.VMEM((L,), jnp.float32),
            pltpu.VMEM((L,), jnp.int32),
            pltpu.VMEM((L,), jnp.int32),
            pltpu.SemaphoreType.DMA,
        ],
        compiler_params=pltpu.CompilerParams(needs_layout_passes=False),
    )(_sc_sample)
    argw, lpvw = sampler(lp, g)
    arg_lst = argw[:, :2]
    return (arg_lst, lpvw[:, 0], lp)


# speedup vs baseline: 1.6687x; 1.6687x over previous
import jax
import jax.numpy as jnp
from jax.experimental import pallas as pl
from jax.experimental.pallas import tpu as pltpu

SIZE = 32
V = SIZE * SIZE
C = 256
B = 64
NB = 8


def _fused_kernel(a_ref, x_ref, g_ref, lp_ref, idx_ref, lpv_ref, xc_ref):
    b = pl.program_id(0)
    rows = []
    for i in range(NB):
        Xi = x_ref[i]
        ai = a_ref[i, :].reshape(1, C)
        rows.append(jax.lax.dot_general(
            ai, Xi, (((1,), (1,)), ((), ()))))
    xc_ref[pl.ds(b * NB, NB), :] = jnp.concatenate(rows, axis=0)

    @pl.when(b == B // NB - 1)
    def _tail():
        xc = xc_ref[...]
        m = jnp.max(xc, axis=1, keepdims=True)
        lse = jnp.log(jnp.sum(jnp.exp(xc - m), axis=1, keepdims=True)) + m
        lp = xc - lse
        lp_ref[...] = lp
        s = lp + g_ref[...]
        smax = jnp.max(s, axis=1, keepdims=True)
        iota = jax.lax.broadcasted_iota(jnp.int32, (B, V), 1)
        idx = jnp.min(jnp.where(s == smax, iota, V), axis=1, keepdims=True)
        idx_ref[...] = idx
        lpv_ref[...] = jnp.sum(jnp.where(iota == idx, lp, 0.0),
                               axis=1, keepdims=True)


def kernel(x, embedded_a):
    xt = x.transpose(0, 2, 3, 1).reshape(B, V, C)
    g = jax.random.gumbel(jax.random.key(42), (B, V), dtype=jnp.float32)
    lp, idx, lpv = pl.pallas_call(
        _fused_kernel,
        grid=(B // NB,),
        in_specs=[
            pl.BlockSpec((NB, C), lambda b: (b, 0)),
            pl.BlockSpec((NB, V, C), lambda b: (b, 0, 0)),
            pl.BlockSpec((B, V), lambda b: (0, 0)),
        ],
        out_specs=[
            pl.BlockSpec((B, V), lambda b: (0, 0)),
            pl.BlockSpec((B, 1), lambda b: (0, 0)),
            pl.BlockSpec((B, 1), lambda b: (0, 0)),
        ],
        out_shape=[
            jax.ShapeDtypeStruct((B, V), jnp.float32),
            jax.ShapeDtypeStruct((B, 1), jnp.int32),
            jax.ShapeDtypeStruct((B, 1), jnp.float32),
        ],
        scratch_shapes=[pltpu.VMEM((B, V), jnp.float32)],
        compiler_params=pltpu.CompilerParams(
            dimension_semantics=("arbitrary",),
        ),
    )(embedded_a, xt, g)
    idx = idx[:, 0]
    arg_lst = jnp.stack([idx % SIZE, idx // SIZE], axis=-1)
    return (arg_lst, lpv[:, 0], lp)
